# Initial kernel scaffold; baseline (speedup 1.0000x reference)
import jax
import jax.numpy as jnp
from jax.experimental import pallas as pl


def kernel(x, W_lin, W_dir):
    raise NotImplementedError("write your pallas kernel here")


# speedup vs baseline: 7.5855x; 7.5855x over previous
import functools

import jax
import jax.numpy as jnp
from jax.experimental import pallas as pl
from jax.experimental.pallas import tpu as pltpu

_K = 20
_NEG = 0.2
_EPS = 1e-6


def _body(xt_ref, wn_ref, wc_ref, out_ref, *, nb, n, k):
    i = pl.program_id(1)
    xt = xt_ref[0]
    xc = xt_ref[0, pl.ds(i * nb, nb), :]
    xx = jnp.sum(xt * xt, axis=1)
    xxc = jnp.sum(xc * xc, axis=1)
    inner = jax.lax.dot_general(
        xc, xt, (((1,), (1,)), ((), ())),
        preferred_element_type=jnp.float32)
    pair = 2.0 * inner - xxc[:, None] - xx[None, :]

    cpd = jnp.dot(xc, wc_ref[...], preferred_element_type=jnp.float32)
    cp = cpd[:, :192]
    cd = cpd[:, 192:]

    iota = jax.lax.broadcasted_iota(jnp.int32, (nb, n), 1)
    m = pair
    acc = jnp.zeros((nb, 192), dtype=jnp.float32)
    for _ in range(k):
        mx = jnp.max(m, axis=1, keepdims=True)
        eq = m == mx
        first = jnp.min(jnp.where(eq, iota, n), axis=1, keepdims=True)
        oh = iota == first
        m = jnp.where(oh, -jnp.inf, m)
        g = jnp.dot(oh.astype(jnp.float32), xt,
                    preferred_element_type=jnp.float32)
        gpd = jnp.dot(g, wn_ref[...], preferred_element_type=jnp.float32)
        p = gpd[:, :192] + cp
        d = gpd[:, 192:] + cd
        p0, p1, p2 = p[:, :64], p[:, 64:128], p[:, 128:]
        d0, d1, d2 = d[:, :64], d[:, 64:128], d[:, 128:]
        dot = p0 * d0 + p1 * d1 + p2 * d2
        dnsq = d0 * d0 + d1 * d1 + d2 * d2
        coef = (1.0 - _NEG) * jnp.minimum(dot, 0.0) / (dnsq + _EPS)
        coef3 = jnp.concatenate([coef, coef, coef], axis=1)
        acc = acc + (p - coef3 * d)
    out_ref[0] = acc * (1.0 / k)


def kernel(x, W_lin, W_dir):
    B, C, V, N = x.shape
    x_t = jnp.transpose(x.reshape(B, C * V, N), (0, 2, 1))
    O = W_lin.shape[0]
    W1 = W_lin[:, :C]
    W2 = W_lin[:, C:]
    D = W_dir @ W_lin
    D1 = D[:, :C]
    D2 = D[:, C:]
    eye = jnp.eye(V, dtype=x.dtype)

    def expand(w):
        return jnp.einsum('oc,vw->cvwo', w, eye).reshape(C * V, V * O)

    Wn = jnp.concatenate([expand(W1), expand(D1)], axis=1)
    Wc = jnp.concatenate([expand(W2 - W1), expand(D2 - D1)], axis=1)

    nb = 256
    body = functools.partial(_body, nb=nb, n=N, k=_K)
    out = pl.pallas_call(
        body,
        grid=(B, N // nb),
        in_specs=[
            pl.BlockSpec((1, N, C * V), lambda b, i: (b, 0, 0)),
            pl.BlockSpec((C * V, 2 * V * O), lambda b, i: (0, 0)),
            pl.BlockSpec((C * V, 2 * V * O), lambda b, i: (0, 0)),
        ],
        out_specs=pl.BlockSpec((1, nb, V * O), lambda b, i: (b, i, 0)),
        out_shape=jax.ShapeDtypeStruct((B, N, V * O), jnp.float32),
    )(x_t, Wn, Wc)
    return out.reshape(B, N, V, O).transpose(0, 3, 2, 1)


# speedup vs baseline: 9.4934x; 1.2515x over previous
import functools

import jax
import jax.numpy as jnp
from jax.experimental import pallas as pl
from jax.experimental.pallas import tpu as pltpu

_K = 20
_NEG = 0.2
_EPS = 1e-6


def _body(xt_ref, wn_ref, wc_ref, out_ref, *, nb, n, k):
    i = pl.program_id(1)
    xt = xt_ref[0]
    xc = xt_ref[0, pl.ds(i * nb, nb), :]
    xx = jnp.sum(xt * xt, axis=1)
    xxc = jnp.sum(xc * xc, axis=1)
    inner = jax.lax.dot_general(
        xc, xt, (((1,), (1,)), ((), ())),
        preferred_element_type=jnp.float32)
    pair = 2.0 * inner - xxc[:, None] - xx[None, :]

    cpd = jnp.dot(xc, wc_ref[...], preferred_element_type=jnp.float32)
    cp = cpd[:, :192]
    cd = cpd[:, 192:]
    cnd = jnp.dot(xc, wn_ref[...], preferred_element_type=jnp.float32)
    cps = cnd[:, :192]
    cds = cnd[:, 192:]

    iota = jax.lax.broadcasted_iota(jnp.int32, (nb, n), 1)
    rows = jax.lax.broadcasted_iota(jnp.int32, (nb, n), 0) + i * nb
    m = jnp.where(iota == rows, -jnp.inf, pair)
    ps = cp + cps
    ds = cd + cds
    s0, s1, s2 = ps[:, :64], ps[:, 64:128], ps[:, 128:]
    t0, t1, t2 = ds[:, :64], ds[:, 64:128], ds[:, 128:]
    sdot = s0 * t0 + s1 * t1 + s2 * t2
    sdn = t0 * t0 + t1 * t1 + t2 * t2
    scoef = (1.0 - _NEG) * jnp.minimum(sdot, 0.0) / (sdn + _EPS)
    acc = ps - jnp.concatenate([scoef, scoef, scoef], axis=1) * ds
    for _ in range(k - 1):
        mx = jnp.max(m, axis=1, keepdims=True)
        first = jnp.min(jnp.where(m == mx, iota, n), axis=1, keepdims=True)
        oh = iota == first
        m = jnp.where(oh, -jnp.inf, m)
        g = jnp.dot(oh.astype(jnp.float32), xt,
                    preferred_element_type=jnp.float32)
        gpd = jnp.dot(g, wn_ref[...], preferred_element_type=jnp.float32)
        p = gpd[:, :192] + cp
        d = gpd[:, 192:] + cd
        p0, p1, p2 = p[:, :64], p[:, 64:128], p[:, 128:]
        d0, d1, d2 = d[:, :64], d[:, 64:128], d[:, 128:]
        dot = p0 * d0 + p1 * d1 + p2 * d2
        dnsq = d0 * d0 + d1 * d1 + d2 * d2
        coef = (1.0 - _NEG) * jnp.minimum(dot, 0.0) / (dnsq + _EPS)
        coef3 = jnp.concatenate([coef, coef, coef], axis=1)
        acc = acc + (p - coef3 * d)
    out_ref[0] = acc * (1.0 / k)


def kernel(x, W_lin, W_dir):
    B, C, V, N = x.shape
    x_t = jnp.transpose(x.reshape(B, C * V, N), (0, 2, 1))
    O = W_lin.shape[0]
    W1 = W_lin[:, :C]
    W2 = W_lin[:, C:]
    D = W_dir @ W_lin
    D1 = D[:, :C]
    D2 = D[:, C:]
    eye = jnp.eye(V, dtype=x.dtype)

    def expand(w):
        return jnp.einsum('oc,vw->cvwo', w, eye).reshape(C * V, V * O)

    Wn = jnp.concatenate([expand(W1), expand(D1)], axis=1)
    Wc = jnp.concatenate([expand(W2 - W1), expand(D2 - D1)], axis=1)

    nb = 1024
    body = functools.partial(_body, nb=nb, n=N, k=_K)
    out = pl.pallas_call(
        body,
        grid=(B, N // nb),
        in_specs=[
            pl.BlockSpec((1, N, C * V), lambda b, i: (b, 0, 0)),
            pl.BlockSpec((C * V, 2 * V * O), lambda b, i: (0, 0)),
            pl.BlockSpec((C * V, 2 * V * O), lambda b, i: (0, 0)),
        ],
        out_specs=pl.BlockSpec((1, nb, V * O), lambda b, i: (b, i, 0)),
        out_shape=jax.ShapeDtypeStruct((B, N, V * O), jnp.float32),
    )(x_t, Wn, Wc)
    return out.reshape(B, N, V, O).transpose(0, 3, 2, 1)


# speedup vs baseline: 11.3272x; 1.1932x over previous
import functools

import jax
import jax.numpy as jnp
from jax.experimental import pallas as pl
from jax.experimental.pallas import tpu as pltpu

_K = 20
_NEG = 0.2
_EPS = 1e-6


def _body(xt_ref, wn_ref, wc_ref, out_ref, *, nb, n, k):
    i = pl.program_id(1)
    xt = xt_ref[0]
    xc = xt_ref[0, pl.ds(i * nb, nb), :]
    xx = jnp.sum(xt * xt, axis=1)
    xxc = jnp.sum(xc * xc, axis=1)
    inner = jax.lax.dot_general(
        xc, xt, (((1,), (1,)), ((), ())),
        preferred_element_type=jnp.float32)
    pair = 2.0 * inner - xxc[:, None] - xx[None, :]

    cpd = jnp.dot(xc, wc_ref[...], preferred_element_type=jnp.float32)
    cp = cpd[:, :192]
    cd = cpd[:, 192:]
    cnd = jnp.dot(xc, wn_ref[...], preferred_element_type=jnp.float32)
    cps = cnd[:, :192]
    cds = cnd[:, 192:]

    iota = jax.lax.broadcasted_iota(jnp.int32, (nb, n), 1)
    rows = jax.lax.broadcasted_iota(jnp.int32, (nb, n), 0) + i * nb
    m = jnp.where(iota == rows, -jnp.inf, pair)
    ps = cp + cps
    ds = cd + cds
    s0, s1, s2 = ps[:, :64], ps[:, 64:128], ps[:, 128:]
    t0, t1, t2 = ds[:, :64], ds[:, 64:128], ds[:, 128:]
    sdot = s0 * t0 + s1 * t1 + s2 * t2
    sdn = t0 * t0 + t1 * t1 + t2 * t2
    scoef = (1.0 - _NEG) * jnp.minimum(sdot, 0.0) / (sdn + _EPS)
    acc = ps - jnp.concatenate([scoef, scoef, scoef], axis=1) * ds
    for _ in range(k - 1):
        mx = jnp.max(m, axis=1, keepdims=True)
        first = jnp.min(jnp.where(m == mx, iota, n), axis=1, keepdims=True)
        oh = iota == first
        m = jnp.where(oh, -jnp.inf, m)
        g = jnp.dot(oh.astype(jnp.float32), xt,
                    preferred_element_type=jnp.float32)
        gpd = jnp.dot(g, wn_ref[...], preferred_element_type=jnp.float32)
        p = gpd[:, :192] + cp
        d = gpd[:, 192:] + cd
        p0, p1, p2 = p[:, :64], p[:, 64:128], p[:, 128:]
        d0, d1, d2 = d[:, :64], d[:, 64:128], d[:, 128:]
        dot = p0 * d0 + p1 * d1 + p2 * d2
        dnsq = d0 * d0 + d1 * d1 + d2 * d2
        coef = (1.0 - _NEG) * jnp.minimum(dot, 0.0) / (dnsq + _EPS)
        coef3 = jnp.concatenate([coef, coef, coef], axis=1)
        acc = acc + (p - coef3 * d)
    out_ref[0] = acc * (1.0 / k)


def kernel(x, W_lin, W_dir):
    B, C, V, N = x.shape
    x_t = jnp.transpose(x.reshape(B, C * V, N), (0, 2, 1))
    O = W_lin.shape[0]
    W1 = W_lin[:, :C]
    W2 = W_lin[:, C:]
    D = W_dir @ W_lin
    D1 = D[:, :C]
    D2 = D[:, C:]
    eye = jnp.eye(V, dtype=x.dtype)

    def expand(w):
        return jnp.einsum('oc,vw->cvwo', w, eye).reshape(C * V, V * O)

    Wn = jnp.concatenate([expand(W1), expand(D1)], axis=1)
    Wc = jnp.concatenate([expand(W2 - W1), expand(D2 - D1)], axis=1)

    nb = 512
    body = functools.partial(_body, nb=nb, n=N, k=_K)
    out = pl.pallas_call(
        body,
        grid=(B, N // nb),
        in_specs=[
            pl.BlockSpec((1, N, C * V), lambda b, i: (b, 0, 0)),
            pl.BlockSpec((C * V, 2 * V * O), lambda b, i: (0, 0)),
            pl.BlockSpec((C * V, 2 * V * O), lambda b, i: (0, 0)),
        ],
        out_specs=pl.BlockSpec((1, nb, V * O), lambda b, i: (b, i, 0)),
        out_shape=jax.ShapeDtypeStruct((B, N, V * O), jnp.float32),
    )(x_t, Wn, Wc)
    return out.reshape(B, N, V, O).transpose(0, 3, 2, 1)
